# baseline (speedup 1.0000x reference)
import jax
import jax.numpy as jnp
from jax import lax
from jax.experimental import pallas as pl
from jax.experimental.pallas import tpu as pltpu
from jax.experimental.pallas import tpu_sc as plsc

B = 16384
D = 128
NC = 2
NS = 16
NW = NC * NS
B_PER_W = B // NW
CHUNK = 128
NCH = B_PER_W // CHUNK
LANES = 16
DG = D // LANES


def _body(tri_hbm, ent_hbm, rel_hbm, out_hbm,
          slab_v, hidx_v, ridx_v, tidx_v,
          hbuf0, rbuf0, tbuf0, hbuf1, rbuf1, tbuf1,
          acc16, out_v, sem0, sem1):
    wid = lax.axis_index("s") * NC + lax.axis_index("c")
    base_row = wid * B_PER_W

    pltpu.sync_copy(tri_hbm.at[pl.ds(base_row * 3, B_PER_W * 3)], slab_v)
    iota = lax.iota(jnp.int32, LANES)

    def split_group(g):
        rows3 = g * (LANES * 3) + iota * 3
        hidx_v[pl.ds(g * LANES, LANES)] = plsc.load_gather(slab_v, [rows3])
        ridx_v[pl.ds(g * LANES, LANES)] = plsc.load_gather(slab_v, [rows3 + 1])
        tidx_v[pl.ds(g * LANES, LANES)] = plsc.load_gather(slab_v, [rows3 + 2])

    bufs = [(hbuf0, rbuf0, tbuf0), (hbuf1, rbuf1, tbuf1)]
    sems = [sem0, sem1]

    def fire(j):
        h, r, t = bufs[j % 2]
        s = sems[j % 2]
        return [
            pltpu.async_copy(ent_hbm.at[hidx_v.at[pl.ds(j * CHUNK, CHUNK)]], h, s),
            pltpu.async_copy(rel_hbm.at[ridx_v.at[pl.ds(j * CHUNK, CHUNK)]], r, s),
            pltpu.async_copy(ent_hbm.at[tidx_v.at[pl.ds(j * CHUNK, CHUNK)]], t, s),
        ]

    gpc = CHUNK // LANES
    for g in range(gpc):
        split_group(g)
    inflight = {0: fire(0)}
    for g in range(gpc, (B_PER_W // LANES)):
        split_group(g)
    inflight[1] = fire(1)

    for j in range(NCH):
        for cp in inflight.pop(j):
            cp.wait()
        hbuf, rbuf, tbuf = bufs[j % 2]

        def group(g, _, j=j, hbuf=hbuf, rbuf=rbuf, tbuf=tbuf):
            base = g * LANES
            for i in range(LANES):
                row = base + i
                acc = (hbuf[row, pl.ds(0, LANES)]
                       * rbuf[row, pl.ds(0, LANES)]
                       * tbuf[row, pl.ds(0, LANES)])
                for dg in range(1, DG):
                    acc = acc + (hbuf[row, pl.ds(dg * LANES, LANES)]
                                 * rbuf[row, pl.ds(dg * LANES, LANES)]
                                 * tbuf[row, pl.ds(dg * LANES, LANES)])
                acc16[pl.ds(i * LANES, LANES)] = acc
            cols = iota * LANES
            score = plsc.load_gather(acc16, [cols])
            for l in range(1, LANES):
                score = score + plsc.load_gather(acc16, [cols + l])
            out_v[pl.ds(j * CHUNK + base, LANES)] = score
            return 0

        lax.fori_loop(0, gpc, group, 0)
        if j + 2 < NCH:
            inflight[j + 2] = fire(j + 2)

    pltpu.sync_copy(out_v, out_hbm.at[pl.ds(base_row, B_PER_W)])


@jax.jit
def _run(tri, ent, rel):
    mesh = plsc.VectorSubcoreMesh(core_axis_name="c", subcore_axis_name="s")
    return pl.kernel(
        _body,
        out_type=jax.ShapeDtypeStruct((B,), jnp.float32),
        mesh=mesh,
        compiler_params=pltpu.CompilerParams(needs_layout_passes=False),
        scratch_types=[
            pltpu.VMEM((B_PER_W * 3,), jnp.int32),
            pltpu.VMEM((B_PER_W,), jnp.int32),
            pltpu.VMEM((B_PER_W,), jnp.int32),
            pltpu.VMEM((B_PER_W,), jnp.int32),
            pltpu.VMEM((CHUNK, D), jnp.float32),
            pltpu.VMEM((CHUNK, D), jnp.float32),
            pltpu.VMEM((CHUNK, D), jnp.float32),
            pltpu.VMEM((CHUNK, D), jnp.float32),
            pltpu.VMEM((CHUNK, D), jnp.float32),
            pltpu.VMEM((CHUNK, D), jnp.float32),
            pltpu.VMEM((LANES * LANES,), jnp.float32),
            pltpu.VMEM((B_PER_W,), jnp.float32),
            pltpu.SemaphoreType.DMA,
            pltpu.SemaphoreType.DMA,
        ],
    )(tri, ent, rel)


def kernel(triplet_idx, entity_embedding, relation_embedding):
    return _run(triplet_idx.astype(jnp.int32).reshape(-1),
                entity_embedding, relation_embedding)
